# baseline (speedup 1.0000x reference)
import jax
import jax.numpy as jnp
from jax.experimental import pallas as pl
from jax.experimental.pallas import tpu as pltpu


def _ncm_body(x_ref, mu_hbm, ck_ref, out_ref, mu_ref, mn_ref, sem):
    i = pl.program_id(0)

    @pl.when(i == 0)
    def _init():
        cp = pltpu.make_async_copy(mu_hbm, mu_ref, sem)
        cp.start()
        cp.wait()
        mu = mu_ref[...]
        ones_row = jnp.ones((1, mu.shape[1]), jnp.float32)
        mn_ref[...] = jax.lax.dot_general(
            ones_row, mu * mu,
            dimension_numbers=(((1,), (1,)), ((), ())),
            preferred_element_type=jnp.float32,
        )

    x = x_ref[...]
    mu = mu_ref[...]
    xn = jnp.sum(x * x, axis=1, keepdims=True)
    g = jax.lax.dot_general(
        x, mu,
        dimension_numbers=(((1,), (1,)), ((), ())),
        preferred_element_type=jnp.float32,
    )
    scores = 2.0 * g - xn - mn_ref[...]
    min_col = jnp.min(scores, axis=1, keepdims=True) - 1.0
    out_ref[...] = jnp.where(ck_ref[...] == 0.0, min_col, scores)


@jax.jit
def kernel(X, muK, cK):
    m, d = X.shape
    k = muK.shape[0]
    ck2 = cK.reshape(1, k)
    bm = 128
    return pl.pallas_call(
        _ncm_body,
        grid=(m // bm,),
        in_specs=[
            pl.BlockSpec((bm, d), lambda i: (i, 0)),
            pl.BlockSpec(memory_space=pltpu.MemorySpace.HBM),
            pl.BlockSpec((1, k), lambda i: (0, 0)),
        ],
        out_specs=pl.BlockSpec((bm, k), lambda i: (i, 0)),
        out_shape=jax.ShapeDtypeStruct((m, k), jnp.float32),
        scratch_shapes=[
            pltpu.MemorySpace.VMEM((k, d), jnp.float32),
            pltpu.MemorySpace.VMEM((1, k), jnp.float32),
            pltpu.SemaphoreType.DMA,
        ],
    )(X, muK, ck2)


# speedup vs baseline: 1.3124x; 1.3124x over previous
import jax
import jax.numpy as jnp
from jax.experimental import pallas as pl
from jax.experimental.pallas import tpu as pltpu


def _ncm_body(x_ref, mu_hbm, ck_ref, out_ref, mu_ref, mn_ref, sem):
    i = pl.program_id(0)

    @pl.when(i == 0)
    def _init():
        cp = pltpu.make_async_copy(mu_hbm, mu_ref, sem)
        cp.start()
        cp.wait()
        mu = mu_ref[...]
        ones_row = jnp.ones((1, mu.shape[1]), jnp.float32)
        mn_ref[...] = jax.lax.dot_general(
            ones_row, mu * mu,
            dimension_numbers=(((1,), (1,)), ((), ())),
            preferred_element_type=jnp.float32,
        )

    x = x_ref[...]
    mu = mu_ref[...]
    xn = jnp.sum(x * x, axis=1, keepdims=True)
    g = jax.lax.dot_general(
        x, mu,
        dimension_numbers=(((1,), (1,)), ((), ())),
        preferred_element_type=jnp.float32,
    )
    scores = 2.0 * g - xn - mn_ref[...]
    min_col = jnp.min(scores, axis=1, keepdims=True) - 1.0
    out_ref[...] = jnp.where(ck_ref[...] == 0.0, min_col, scores)


@jax.jit
def kernel(X, muK, cK):
    m, d = X.shape
    k = muK.shape[0]
    ck2 = cK.reshape(1, k)
    bm = 512
    return pl.pallas_call(
        _ncm_body,
        grid=(m // bm,),
        in_specs=[
            pl.BlockSpec((bm, d), lambda i: (i, 0)),
            pl.BlockSpec(memory_space=pltpu.MemorySpace.HBM),
            pl.BlockSpec((1, k), lambda i: (0, 0)),
        ],
        out_specs=pl.BlockSpec((bm, k), lambda i: (i, 0)),
        out_shape=jax.ShapeDtypeStruct((m, k), jnp.float32),
        scratch_shapes=[
            pltpu.MemorySpace.VMEM((k, d), jnp.float32),
            pltpu.MemorySpace.VMEM((1, k), jnp.float32),
            pltpu.SemaphoreType.DMA,
        ],
    )(X, muK, ck2)
